# baseline (speedup 1.0000x reference)
import functools

import jax
import jax.numpy as jnp
from jax import lax
from jax.experimental import pallas as pl
from jax.experimental.pallas import tpu as pltpu
from jax.experimental.pallas import tpu_sc as plsc

F = 26
B = 16384
FIELD_DIM = 100000
L = 16
NC = 2
NS = 16
NW = NC * NS
PER_W = B // NW
E = PER_W * F
NCHUNK = PER_W // L


def _make_kernel():
    mesh = plsc.VectorSubcoreMesh(
        core_axis_name="c", subcore_axis_name="s", num_cores=NC, num_subcores=NS
    )

    @functools.partial(
        pl.kernel,
        mesh=mesh,
        out_type=jax.ShapeDtypeStruct((B,), jnp.float32),
        compiler_params=pltpu.CompilerParams(needs_layout_passes=False),
        scratch_types=[
            pltpu.VMEM((E,), jnp.int32),
            pltpu.VMEM((E,), jnp.int32),
            pltpu.VMEM((E,), jnp.float32),
            pltpu.VMEM((PER_W,), jnp.float32),
            pltpu.VMEM((L,), jnp.float32),
            pltpu.SemaphoreType.DMA,
        ],
    )
    def k(x_hbm, table_hbm, bias_hbm, out_hbm, xv, idxv, rowsv, outv, biasv, sem):
        wid = lax.axis_index("s") * NC + lax.axis_index("c")
        base = wid * PER_W
        pltpu.sync_copy(x_hbm.at[pl.ds(base * F, E)], xv)
        pltpu.sync_copy(bias_hbm, biasv)

        lanes26 = lax.iota(jnp.int32, L) * F

        def build(c, carry):
            src_base = c * (L * F)
            for f in range(F):
                vals = plsc.load_gather(xv, [lanes26 + (src_base + f)])
                idxv[pl.ds(f * PER_W + c * L, L)] = vals + f * FIELD_DIM
            return carry

        lax.fori_loop(0, NCHUNK, build, 0)

        pltpu.async_copy(table_hbm.at[idxv], rowsv, sem).wait()

        bias_vec = biasv[...]

        def reduce(c, carry):
            acc = bias_vec
            for f in range(F):
                acc = acc + rowsv[pl.ds(f * PER_W + c * L, L)]
            outv[pl.ds(c * L, L)] = acc
            return carry

        lax.fori_loop(0, NCHUNK, reduce, 0)
        pltpu.sync_copy(outv, out_hbm.at[pl.ds(base, PER_W)])

    return k


_sc_kernel = _make_kernel()


@jax.jit
def kernel(x, table, bias):
    xf = x.reshape(-1)
    tf = table.T.reshape(-1)
    bb = jnp.broadcast_to(bias.reshape(-1)[:1], (L,))
    out = _sc_kernel(xf, tf, bb)
    return out.reshape(B, 1)


# speedup vs baseline: 1.0014x; 1.0014x over previous
import functools

import jax
import jax.numpy as jnp
from jax import lax
from jax.experimental import pallas as pl
from jax.experimental.pallas import tpu as pltpu
from jax.experimental.pallas import tpu_sc as plsc

F = 26
B = 16384
FIELD_DIM = 100000
L = 16
NC = 2
NS = 16
NW = NC * NS
PER_W = B // NW
E = PER_W * F
NCHUNK = PER_W // L


def _make_kernel():
    mesh = plsc.VectorSubcoreMesh(
        core_axis_name="c", subcore_axis_name="s", num_cores=NC, num_subcores=NS
    )

    @functools.partial(
        pl.kernel,
        mesh=mesh,
        out_type=jax.ShapeDtypeStruct((B,), jnp.float32),
        compiler_params=pltpu.CompilerParams(needs_layout_passes=False),
        scratch_types=[
            pltpu.VMEM((E,), jnp.int32),
            pltpu.VMEM((E,), jnp.int32),
            pltpu.VMEM((E,), jnp.float32),
            pltpu.VMEM((PER_W,), jnp.float32),
            pltpu.VMEM((L,), jnp.float32),
            pltpu.SemaphoreType.DMA,
        ],
    )
    def k(x_hbm, table_hbm, bias_hbm, out_hbm, xv, idxv, rowsv, outv, biasv, sem):
        wid = lax.axis_index("s") * NC + lax.axis_index("c")
        base = wid * PER_W
        pltpu.sync_copy(x_hbm.at[pl.ds(base * F, E)], xv)
        pltpu.sync_copy(bias_hbm, biasv)

        lanes26 = lax.iota(jnp.int32, L) * F

        def build(c, carry):
            src_base = c * (L * F)
            for f in range(F):
                vals = plsc.load_gather(xv, [lanes26 + (src_base + f)])
                idxv[pl.ds(f * PER_W + c * L, L)] = vals + f * FIELD_DIM
            return carry

        lax.fori_loop(0, NCHUNK, build, 0)

        pltpu.async_copy(table_hbm.at[idxv], rowsv, sem).wait()

        bias_vec = biasv[...]

        def reduce(c, carry):
            acc = bias_vec
            for f in range(F):
                acc = acc + rowsv[pl.ds(f * PER_W + c * L, L)]
            outv[pl.ds(c * L, L)] = acc
            return carry

        lax.fori_loop(0, NCHUNK, reduce, 0)
        pltpu.sync_copy(outv, out_hbm.at[pl.ds(base, PER_W)])

    return k


_sc_kernel = _make_kernel()


@jax.jit
def kernel(x, table, bias):
    xf = x.reshape(-1)
    tf = (table + bias[:1, :1] * (1.0 / F)).reshape(-1)
    bb = jnp.zeros((L,), jnp.float32)
    out = _sc_kernel(xf, tf, bb)
    return out.reshape(B, 1)


# speedup vs baseline: 1.1240x; 1.1224x over previous
import functools

import jax
import jax.numpy as jnp
from jax import lax
from jax.experimental import pallas as pl
from jax.experimental.pallas import tpu as pltpu
from jax.experimental.pallas import tpu_sc as plsc

F = 26
B = 16384
FIELD_DIM = 100000
L = 16
NC = 2
NS = 16
NW = NC * NS
PER_W = B // NW
E = PER_W * F
NCHUNK = PER_W // L


def _make_kernel():
    mesh = plsc.VectorSubcoreMesh(
        core_axis_name="c", subcore_axis_name="s", num_cores=NC, num_subcores=NS
    )

    @functools.partial(
        pl.kernel,
        mesh=mesh,
        out_type=jax.ShapeDtypeStruct((B,), jnp.float32),
        compiler_params=pltpu.CompilerParams(needs_layout_passes=False),
        scratch_types=[
            pltpu.VMEM((E,), jnp.int32),
            pltpu.VMEM((E,), jnp.float32),
            pltpu.VMEM((PER_W,), jnp.float32),
            pltpu.SemaphoreType.DMA,
        ],
    )
    def k(xt_hbm, table_hbm, out_hbm, idxv, rowsv, outv, sem):
        wid = lax.axis_index("s") * NC + lax.axis_index("c")
        base = wid * PER_W

        descs = [
            pltpu.async_copy(
                xt_hbm.at[f, pl.ds(base, PER_W)],
                idxv.at[pl.ds(f * PER_W, PER_W)],
                sem,
            )
            for f in range(F)
        ]
        for d in descs:
            d.wait()

        def build(c, carry):
            off = (c // NCHUNK) * FIELD_DIM
            idxv[pl.ds(c * L, L)] = idxv[pl.ds(c * L, L)] + off
            return carry

        lax.fori_loop(0, E // L, build, 0)

        pltpu.async_copy(table_hbm.at[idxv], rowsv, sem).wait()

        def reduce(c, carry):
            acc = rowsv[pl.ds(c * L, L)]
            for f in range(1, F):
                acc = acc + rowsv[pl.ds(f * PER_W + c * L, L)]
            outv[pl.ds(c * L, L)] = acc
            return carry

        lax.fori_loop(0, NCHUNK, reduce, 0)
        pltpu.sync_copy(outv, out_hbm.at[pl.ds(base, PER_W)])

    return k


_sc_kernel = _make_kernel()


@jax.jit
def kernel(x, table, bias):
    xt = x.T
    tf = (table + bias[:1, :1] * (1.0 / F)).reshape(-1)
    out = _sc_kernel(xt, tf)
    return out.reshape(B, 1)


# speedup vs baseline: 2.7028x; 2.4046x over previous
import functools

import jax
import jax.numpy as jnp
from jax import lax
from jax.experimental import pallas as pl
from jax.experimental.pallas import tpu as pltpu
from jax.experimental.pallas import tpu_sc as plsc

F = 26
B = 16384
FIELD_DIM = 100000
TABLE_N = F * FIELD_DIM
TABLE_PAD = 2600960
L = 16
NC = 2
NS = 16
NW = NC * NS
PER_W = B // NW
E = PER_W * F
NCHUNK = PER_W // L


def _make_kernel():
    mesh = plsc.VectorSubcoreMesh(
        core_axis_name="c", subcore_axis_name="s", num_cores=NC, num_subcores=NS
    )

    @functools.partial(
        pl.kernel,
        mesh=mesh,
        out_type=jax.ShapeDtypeStruct((B,), jnp.float32),
        compiler_params=pltpu.CompilerParams(needs_layout_passes=False),
        scratch_types=[
            pltpu.VMEM((E,), jnp.int32),
            pltpu.VMEM((E,), jnp.float32),
            pltpu.VMEM((PER_W,), jnp.float32),
            pltpu.SemaphoreType.DMA,
        ],
    )
    def k(xt_hbm, table_hbm, out_hbm, idxv, rowsv, outv, sem):
        wid = lax.axis_index("s") * NC + lax.axis_index("c")
        base = wid * PER_W

        descs = [
            pltpu.async_copy(
                xt_hbm.at[f, pl.ds(base, PER_W)],
                idxv.at[pl.ds(f * PER_W, PER_W)],
                sem,
            )
            for f in range(F)
        ]
        for d in descs:
            d.wait()

        def build(c, carry):
            off = (c // NCHUNK) * FIELD_DIM
            idxv[pl.ds(c * L, L)] = idxv[pl.ds(c * L, L)] + off
            return carry

        lax.fori_loop(0, E // L, build, 0)

        pltpu.async_copy(table_hbm.at[idxv], rowsv, sem).wait()

        def reduce(c, carry):
            acc = rowsv[pl.ds(c * L, L)]
            for f in range(1, F):
                acc = acc + rowsv[pl.ds(f * PER_W + c * L, L)]
            outv[pl.ds(c * L, L)] = acc
            return carry

        lax.fori_loop(0, NCHUNK, reduce, 0)
        pltpu.sync_copy(outv, out_hbm.at[pl.ds(base, PER_W)])

    return k


_sc_kernel = _make_kernel()


@jax.jit
def kernel(x, table, bias):
    xt = x.T
    tb = table + bias[:1, :1] * (1.0 / F)
    tf = jnp.concatenate(
        [tb, jnp.zeros((TABLE_PAD - TABLE_N, 1), jnp.float32)], axis=0
    ).reshape(-1)
    out = _sc_kernel(xt, tf)
    return out.reshape(B, 1)


# speedup vs baseline: 2.9827x; 1.1036x over previous
import functools

import jax
import jax.numpy as jnp
from jax import lax
from jax.experimental import pallas as pl
from jax.experimental.pallas import tpu as pltpu
from jax.experimental.pallas import tpu_sc as plsc

F = 26
B = 16384
FIELD_DIM = 100000
TABLE_N = F * FIELD_DIM
TABLE_PAD = 2600960
L = 16
NC = 2
NS = 16
NW = NC * NS
PER_W = B // NW
E = PER_W * F
NCHUNK = PER_W // L


def _make_kernel():
    mesh = plsc.VectorSubcoreMesh(
        core_axis_name="c", subcore_axis_name="s", num_cores=NC, num_subcores=NS
    )

    @functools.partial(
        pl.kernel,
        mesh=mesh,
        out_type=jax.ShapeDtypeStruct((B,), jnp.float32),
        compiler_params=pltpu.CompilerParams(needs_layout_passes=False),
        scratch_types=[
            pltpu.VMEM((E,), jnp.int32),
            pltpu.VMEM((E,), jnp.float32),
            pltpu.VMEM((PER_W,), jnp.float32),
            pltpu.VMEM((L,), jnp.float32),
            pltpu.SemaphoreType.DMA,
        ],
    )
    def k(xt_hbm, table_hbm, bias_hbm, out_hbm, idxv, rowsv, outv, biasv, sem):
        wid = lax.axis_index("s") * NC + lax.axis_index("c")
        base = wid * PER_W

        descs = [
            pltpu.async_copy(
                xt_hbm.at[f, pl.ds(base, PER_W)],
                idxv.at[pl.ds(f * PER_W, PER_W)],
                sem,
            )
            for f in range(F)
        ]
        for d in descs:
            d.wait()

        def build(c, carry):
            off = (c // NCHUNK) * FIELD_DIM
            idxv[pl.ds(c * L, L)] = idxv[pl.ds(c * L, L)] + off
            return carry

        lax.fori_loop(0, E // L, build, 0)

        pltpu.sync_copy(bias_hbm, biasv)

        pltpu.async_copy(table_hbm.at[idxv], rowsv, sem).wait()

        bias_vec = biasv[...]

        def reduce(c, carry):
            acc = bias_vec + rowsv[pl.ds(c * L, L)]
            for f in range(1, F):
                acc = acc + rowsv[pl.ds(f * PER_W + c * L, L)]
            outv[pl.ds(c * L, L)] = acc
            return carry

        lax.fori_loop(0, NCHUNK, reduce, 0)
        pltpu.sync_copy(outv, out_hbm.at[pl.ds(base, PER_W)])

    return k


_sc_kernel = _make_kernel()


@jax.jit
def kernel(x, table, bias):
    xt = x.T
    tf = jnp.concatenate(
        [table, jnp.zeros((TABLE_PAD - TABLE_N, 1), jnp.float32)], axis=0
    ).reshape(-1)
    bb = jnp.broadcast_to(bias.reshape(-1)[:1], (L,))
    out = _sc_kernel(xt, tf, bb)
    return out.reshape(B, 1)


# speedup vs baseline: 3.1464x; 1.0549x over previous
import functools

import jax
import jax.numpy as jnp
from jax import lax
from jax.experimental import pallas as pl
from jax.experimental.pallas import tpu as pltpu
from jax.experimental.pallas import tpu_sc as plsc

F = 26
B = 16384
FIELD_DIM = 100000
TABLE_N = F * FIELD_DIM
TABLE_PAD = 2600960
L = 16
NC = 2
NS = 16
NW = NC * NS
PER_W = B // NW
E = PER_W * F
NCHUNK = PER_W // L
FH = F // 2
EH = E // 2


def _make_kernel():
    mesh = plsc.VectorSubcoreMesh(
        core_axis_name="c", subcore_axis_name="s", num_cores=NC, num_subcores=NS
    )

    @functools.partial(
        pl.kernel,
        mesh=mesh,
        out_type=jax.ShapeDtypeStruct((B,), jnp.float32),
        compiler_params=pltpu.CompilerParams(needs_layout_passes=False),
        scratch_types=[
            pltpu.VMEM((E,), jnp.int32),
            pltpu.VMEM((E,), jnp.float32),
            pltpu.VMEM((PER_W,), jnp.float32),
            pltpu.VMEM((L,), jnp.float32),
            pltpu.SemaphoreType.DMA,
            pltpu.SemaphoreType.DMA,
            pltpu.SemaphoreType.DMA,
        ],
    )
    def k(xt_hbm, table_hbm, bias_hbm, out_hbm,
          idxv, rowsv, outv, biasv, sem, semA, semB):
        wid = lax.axis_index("s") * NC + lax.axis_index("c")
        base = wid * PER_W

        descs = [
            pltpu.async_copy(
                xt_hbm.at[f, pl.ds(base, PER_W)],
                idxv.at[pl.ds(f * PER_W, PER_W)],
                sem,
            )
            for f in range(F)
        ]
        pltpu.sync_copy(bias_hbm, biasv)

        def build(c, carry):
            off = (c // NCHUNK) * FIELD_DIM
            idxv[pl.ds(c * L, L)] = idxv[pl.ds(c * L, L)] + off
            return carry

        for d in descs[:FH]:
            d.wait()
        lax.fori_loop(0, EH // L, build, 0)
        g1 = pltpu.async_copy(
            table_hbm.at[idxv.at[pl.ds(0, EH)]], rowsv.at[pl.ds(0, EH)], semA
        )

        for d in descs[FH:]:
            d.wait()
        lax.fori_loop(EH // L, E // L, build, 0)
        g2 = pltpu.async_copy(
            table_hbm.at[idxv.at[pl.ds(EH, EH)]], rowsv.at[pl.ds(EH, EH)], semB
        )

        bias_vec = biasv[...]

        g1.wait()

        def reduce1(c, carry):
            acc = bias_vec + rowsv[pl.ds(c * L, L)]
            for f in range(1, FH):
                acc = acc + rowsv[pl.ds(f * PER_W + c * L, L)]
            outv[pl.ds(c * L, L)] = acc
            return carry

        lax.fori_loop(0, NCHUNK, reduce1, 0)

        g2.wait()

        def reduce2(c, carry):
            acc = outv[pl.ds(c * L, L)]
            for f in range(FH, F):
                acc = acc + rowsv[pl.ds(f * PER_W + c * L, L)]
            outv[pl.ds(c * L, L)] = acc
            return carry

        lax.fori_loop(0, NCHUNK, reduce2, 0)
        pltpu.sync_copy(outv, out_hbm.at[pl.ds(base, PER_W)])

    return k


_sc_kernel = _make_kernel()


@jax.jit
def kernel(x, table, bias):
    xt = x.T
    tf = jnp.concatenate(
        [table, jnp.zeros((TABLE_PAD - TABLE_N, 1), jnp.float32)], axis=0
    ).reshape(-1)
    bb = jnp.broadcast_to(bias.reshape(-1)[:1], (L,))
    out = _sc_kernel(xt, tf, bb)
    return out.reshape(B, 1)


# speedup vs baseline: 3.1962x; 1.0158x over previous
import functools

import jax
import jax.numpy as jnp
from jax import lax
from jax.experimental import pallas as pl
from jax.experimental.pallas import tpu as pltpu
from jax.experimental.pallas import tpu_sc as plsc

F = 26
B = 16384
FIELD_DIM = 100000
TABLE_N = F * FIELD_DIM
TABLE_PAD = 2600960
L = 16
NC = 2
NS = 16
NW = NC * NS
PER_W = B // NW
E = PER_W * F
NCHUNK = PER_W // L
FH = F // 2
EH = E // 2

_MESH = plsc.VectorSubcoreMesh(
    core_axis_name="c", subcore_axis_name="s", num_cores=NC, num_subcores=NS
)


def _worker_base():
    return (lax.axis_index("s") * NC + lax.axis_index("c")) * PER_W


@functools.partial(
    pl.kernel,
    mesh=_MESH,
    out_type=jax.ShapeDtypeStruct((B * F,), jnp.int32),
    compiler_params=pltpu.CompilerParams(needs_layout_passes=False),
    scratch_types=[
        pltpu.VMEM((E,), jnp.int32),
        pltpu.SemaphoreType.DMA,
    ],
)
def _sc_build(xt_hbm, idx_hbm, idxv, sem):
    base = _worker_base()

    descs = [
        pltpu.async_copy(
            xt_hbm.at[f, pl.ds(base, PER_W)],
            idxv.at[pl.ds(f * PER_W, PER_W)],
            sem,
        )
        for f in range(F)
    ]
    for d in descs:
        d.wait()

    def build(c, carry):
        off = (c // NCHUNK) * FIELD_DIM
        idxv[pl.ds(c * L, L)] = idxv[pl.ds(c * L, L)] + off
        return carry

    lax.fori_loop(0, E // L, build, 0)
    pltpu.sync_copy(idxv, idx_hbm.at[pl.ds(base * F, E)])


@functools.partial(
    pl.kernel,
    mesh=_MESH,
    out_type=jax.ShapeDtypeStruct((B,), jnp.float32),
    compiler_params=pltpu.CompilerParams(needs_layout_passes=False),
    scratch_types=[
        pltpu.VMEM((E,), jnp.int32),
        pltpu.VMEM((E,), jnp.float32),
        pltpu.VMEM((PER_W,), jnp.float32),
        pltpu.VMEM((L,), jnp.float32),
        pltpu.SemaphoreType.DMA,
        pltpu.SemaphoreType.DMA,
        pltpu.SemaphoreType.DMA,
    ],
)
def _sc_main(idx_hbm, table_hbm, bias_hbm, out_hbm,
             idxv, rowsv, outv, biasv, sem, semA, semB):
    base = _worker_base()

    d1 = pltpu.async_copy(
        idx_hbm.at[pl.ds(base * F, EH)], idxv.at[pl.ds(0, EH)], sem
    )
    d2 = pltpu.async_copy(
        idx_hbm.at[pl.ds(base * F + EH, EH)], idxv.at[pl.ds(EH, EH)], sem
    )
    pltpu.sync_copy(bias_hbm, biasv)
    d1.wait()
    g1 = pltpu.async_copy(
        table_hbm.at[idxv.at[pl.ds(0, EH)]], rowsv.at[pl.ds(0, EH)], semA
    )
    d2.wait()
    g2 = pltpu.async_copy(
        table_hbm.at[idxv.at[pl.ds(EH, EH)]], rowsv.at[pl.ds(EH, EH)], semB
    )

    bias_vec = biasv[...]

    g1.wait()

    def reduce1(c, carry):
        acc = bias_vec + rowsv[pl.ds(c * L, L)]
        for f in range(1, FH):
            acc = acc + rowsv[pl.ds(f * PER_W + c * L, L)]
        outv[pl.ds(c * L, L)] = acc
        return carry

    lax.fori_loop(0, NCHUNK, reduce1, 0)

    g2.wait()

    def reduce2(c, carry):
        acc = outv[pl.ds(c * L, L)]
        for f in range(FH, F):
            acc = acc + rowsv[pl.ds(f * PER_W + c * L, L)]
        outv[pl.ds(c * L, L)] = acc
        return carry

    lax.fori_loop(0, NCHUNK, reduce2, 0)
    pltpu.sync_copy(outv, out_hbm.at[pl.ds(base, PER_W)])


@jax.jit
def kernel(x, table, bias):
    xt = x.T
    tf = jnp.concatenate(
        [table, jnp.zeros((TABLE_PAD - TABLE_N, 1), jnp.float32)], axis=0
    ).reshape(-1)
    bb = jnp.broadcast_to(bias.reshape(-1)[:1], (L,))
    idx_all = _sc_build(xt)
    out = _sc_main(idx_all, tf, bb)
    return out.reshape(B, 1)


# speedup vs baseline: 3.2281x; 1.0100x over previous
import functools

import jax
import jax.numpy as jnp
from jax import lax
from jax.experimental import pallas as pl
from jax.experimental.pallas import tpu as pltpu
from jax.experimental.pallas import tpu_sc as plsc

F = 26
B = 16384
FIELD_DIM = 100000
TABLE_N = F * FIELD_DIM
TABLE_PAD = 2600960
L = 16
NC = 2
NS = 16
NW = NC * NS
PER_W = B // NW
E = PER_W * F
NCHUNK = PER_W // L
FH = F // 2
EH = E // 2
FQ = 2

_MESH = plsc.VectorSubcoreMesh(
    core_axis_name="c", subcore_axis_name="s", num_cores=NC, num_subcores=NS
)


def _worker_base():
    return (lax.axis_index("s") * NC + lax.axis_index("c")) * PER_W


@functools.partial(
    pl.kernel,
    mesh=_MESH,
    out_type=jax.ShapeDtypeStruct((B * F,), jnp.int32),
    compiler_params=pltpu.CompilerParams(needs_layout_passes=False),
    scratch_types=[
        pltpu.VMEM((E,), jnp.int32),
        pltpu.SemaphoreType.DMA,
    ],
)
def _sc_build(xt_hbm, idx_hbm, idxv, sem):
    base = _worker_base()

    descs = [
        pltpu.async_copy(
            xt_hbm.at[f, pl.ds(base, PER_W)],
            idxv.at[pl.ds(f * PER_W, PER_W)],
            sem,
        )
        for f in range(F)
    ]
    for d in descs:
        d.wait()

    def build(c, carry):
        off = (c // NCHUNK) * FIELD_DIM
        idxv[pl.ds(c * L, L)] = idxv[pl.ds(c * L, L)] + off
        return carry

    lax.fori_loop(0, E // L, build, 0)
    pltpu.sync_copy(idxv, idx_hbm.at[pl.ds(base * F, E)])


@functools.partial(
    pl.kernel,
    mesh=_MESH,
    out_type=jax.ShapeDtypeStruct((B,), jnp.float32),
    compiler_params=pltpu.CompilerParams(needs_layout_passes=False),
    scratch_types=[
        pltpu.VMEM((E,), jnp.int32),
        pltpu.VMEM((E,), jnp.float32),
        pltpu.VMEM((PER_W,), jnp.float32),
        pltpu.VMEM((L,), jnp.float32),
        pltpu.SemaphoreType.DMA,
        [pltpu.SemaphoreType.DMA] * (F // FQ),
    ],
)
def _sc_main(idx_hbm, table_hbm, bias_hbm, out_hbm,
             idxv, rowsv, outv, biasv, sem, gsems):
    base = _worker_base()
    NQ = F // FQ
    EQ = FQ * PER_W

    dloads = [
        pltpu.async_copy(
            idx_hbm.at[pl.ds(base * F + q * EQ, EQ)],
            idxv.at[pl.ds(q * EQ, EQ)],
            sem,
        )
        for q in range(NQ)
    ]
    pltpu.sync_copy(bias_hbm, biasv)

    gathers = []
    for q in range(NQ):
        dloads[q].wait()
        gathers.append(
            pltpu.async_copy(
                table_hbm.at[idxv.at[pl.ds(q * EQ, EQ)]],
                rowsv.at[pl.ds(q * EQ, EQ)],
                gsems[q],
            )
        )

    bias_vec = biasv[...]

    for q in range(NQ):
        gathers[q].wait()

        def reduce_q(c, carry, q=q):
            if q == 0:
                acc = bias_vec + rowsv[pl.ds(c * L, L)]
                flo = 1
            else:
                acc = outv[pl.ds(c * L, L)]
                flo = q * FQ
            for f in range(flo, (q + 1) * FQ):
                acc = acc + rowsv[pl.ds(f * PER_W + c * L, L)]
            outv[pl.ds(c * L, L)] = acc
            return carry

        lax.fori_loop(0, NCHUNK, reduce_q, 0)

    pltpu.sync_copy(outv, out_hbm.at[pl.ds(base, PER_W)])


@jax.jit
def kernel(x, table, bias):
    xt = x.T
    tf = jnp.concatenate(
        [table, jnp.zeros((TABLE_PAD - TABLE_N, 1), jnp.float32)], axis=0
    ).reshape(-1)
    bb = jnp.broadcast_to(bias.reshape(-1)[:1], (L,))
    idx_all = _sc_build(xt)
    out = _sc_main(idx_all, tf, bb)
    return out.reshape(B, 1)


# speedup vs baseline: 3.2403x; 1.0038x over previous
import functools

import jax
import jax.numpy as jnp
from jax import lax
from jax.experimental import pallas as pl
from jax.experimental.pallas import tpu as pltpu
from jax.experimental.pallas import tpu_sc as plsc

F = 26
B = 16384
FIELD_DIM = 100000
TABLE_N = F * FIELD_DIM
TABLE_PAD = 2600960
L = 16
NC = 2
NS = 16
NW = NC * NS
PER_W = B // NW
E = PER_W * F
NCHUNK = PER_W // L
FH = F // 2
EH = E // 2
FQ = 2

_MESH = plsc.VectorSubcoreMesh(
    core_axis_name="c", subcore_axis_name="s", num_cores=NC, num_subcores=NS
)


def _worker_base():
    return (lax.axis_index("s") * NC + lax.axis_index("c")) * PER_W


@functools.partial(
    pl.kernel,
    mesh=_MESH,
    out_type=jax.ShapeDtypeStruct((B * F,), jnp.int32),
    compiler_params=pltpu.CompilerParams(needs_layout_passes=False),
    scratch_types=[
        pltpu.VMEM((E,), jnp.int32),
        pltpu.SemaphoreType.DMA,
    ],
)
def _sc_build(xt_hbm, idx_hbm, idxv, sem):
    base = _worker_base()

    descs = [
        pltpu.async_copy(
            xt_hbm.at[f, pl.ds(base, PER_W)],
            idxv.at[pl.ds(f * PER_W, PER_W)],
            sem,
        )
        for f in range(F)
    ]
    for d in descs:
        d.wait()

    def build(c, carry):
        off = (c // NCHUNK) * FIELD_DIM
        idxv[pl.ds(c * L, L)] = idxv[pl.ds(c * L, L)] + off
        return carry

    lax.fori_loop(0, E // L, build, 0)
    pltpu.sync_copy(idxv, idx_hbm.at[pl.ds(base * F, E)])


@functools.partial(
    pl.kernel,
    mesh=_MESH,
    out_type=jax.ShapeDtypeStruct((B,), jnp.float32),
    compiler_params=pltpu.CompilerParams(needs_layout_passes=False),
    scratch_types=[
        pltpu.VMEM((E,), jnp.int32),
        pltpu.VMEM((E,), jnp.float32),
        pltpu.VMEM((PER_W,), jnp.float32),
        pltpu.VMEM((L,), jnp.float32),
        pltpu.SemaphoreType.DMA,
        [pltpu.SemaphoreType.DMA] * (F // FQ),
    ],
)
def _sc_main(idx_hbm, table_hbm, bias_hbm, out_hbm,
             idxv, rowsv, outv, biasv, sem, gsems):
    base = _worker_base()
    NQ = F // FQ
    EQ = FQ * PER_W

    dloads = [
        pltpu.async_copy(
            idx_hbm.at[pl.ds(base * F + q * EQ, EQ)],
            idxv.at[pl.ds(q * EQ, EQ)],
            sem,
        )
        for q in range(NQ)
    ]
    pltpu.sync_copy(bias_hbm, biasv)

    gathers = []
    for q in range(NQ):
        dloads[q].wait()
        gathers.append(
            pltpu.async_copy(
                table_hbm.at[idxv.at[pl.ds(q * EQ, EQ)]],
                rowsv.at[pl.ds(q * EQ, EQ)],
                gsems[q],
            )
        )

    bias_vec = biasv[...]

    for q in range(NQ):
        gathers[q].wait()

        def reduce_q(c, carry, q=q):
            if q == 0:
                acc = bias_vec + rowsv[pl.ds(c * L, L)]
                flo = 1
            else:
                acc = outv[pl.ds(c * L, L)]
                flo = q * FQ
            for f in range(flo, (q + 1) * FQ):
                acc = acc + rowsv[pl.ds(f * PER_W + c * L, L)]
            outv[pl.ds(c * L, L)] = acc
            return carry

        lax.fori_loop(0, NCHUNK, reduce_q, 0)

    pltpu.sync_copy(outv, out_hbm.at[pl.ds(base, PER_W)])


@jax.jit
def kernel(x, table, bias):
    xt = x.T
    tf = lax.dynamic_update_slice(
        jnp.zeros((TABLE_PAD, 1), jnp.float32), table, (0, 0)
    ).reshape(-1)
    bb = jnp.broadcast_to(bias.reshape(-1)[:1], (L,))
    idx_all = _sc_build(xt)
    out = _sc_main(idx_all, tf, bb)
    return out.reshape(B, 1)


# speedup vs baseline: 3.2795x; 1.0121x over previous
import functools

import jax
import jax.numpy as jnp
from jax import lax
from jax.experimental import pallas as pl
from jax.experimental.pallas import tpu as pltpu
from jax.experimental.pallas import tpu_sc as plsc

F = 26
B = 16384
FIELD_DIM = 100000
TABLE_N = F * FIELD_DIM
TABLE_PAD = 2600960
L = 16
NC = 2
NS = 16
NW = NC * NS
PER_W = B // NW
E = PER_W * F
NCHUNK = PER_W // L
FH = F // 2
EH = E // 2
FQ = 2

_MESH = plsc.VectorSubcoreMesh(
    core_axis_name="c", subcore_axis_name="s", num_cores=NC, num_subcores=NS
)


def _worker_base():
    return (lax.axis_index("s") * NC + lax.axis_index("c")) * PER_W


@functools.partial(
    pl.kernel,
    mesh=_MESH,
    out_type=jax.ShapeDtypeStruct((B * F,), jnp.int32),
    compiler_params=pltpu.CompilerParams(needs_layout_passes=False),
    scratch_types=[
        pltpu.VMEM((E,), jnp.int32),
        pltpu.SemaphoreType.DMA,
    ],
)
def _sc_build(xt_hbm, idx_hbm, idxv, sem):
    base = _worker_base()

    descs = [
        pltpu.async_copy(
            xt_hbm.at[f, pl.ds(base, PER_W)],
            idxv.at[pl.ds(f * PER_W, PER_W)],
            sem,
        )
        for f in range(F)
    ]
    for d in descs:
        d.wait()

    def build(c, carry):
        off = (c // NCHUNK) * FIELD_DIM
        idxv[pl.ds(c * L, L)] = idxv[pl.ds(c * L, L)] + off
        return carry

    lax.fori_loop(0, E // L, build, 0)
    pltpu.sync_copy(idxv, idx_hbm.at[pl.ds(base * F, E)])


@functools.partial(
    pl.kernel,
    mesh=_MESH,
    out_type=jax.ShapeDtypeStruct((B,), jnp.float32),
    compiler_params=pltpu.CompilerParams(needs_layout_passes=False),
    scratch_types=[
        pltpu.VMEM((E,), jnp.int32),
        pltpu.VMEM((E,), jnp.float32),
        pltpu.VMEM((PER_W,), jnp.float32),
        pltpu.VMEM((1, 1), jnp.float32),
        pltpu.SemaphoreType.DMA,
        [pltpu.SemaphoreType.DMA] * (F // FQ),
    ],
)
def _sc_main(idx_hbm, table_hbm, bias_hbm, out_hbm,
             idxv, rowsv, outv, biasv, sem, gsems):
    base = _worker_base()
    NQ = F // FQ
    EQ = FQ * PER_W

    dloads = [
        pltpu.async_copy(
            idx_hbm.at[pl.ds(base * F + q * EQ, EQ)],
            idxv.at[pl.ds(q * EQ, EQ)],
            sem,
        )
        for q in range(NQ)
    ]
    pltpu.sync_copy(bias_hbm, biasv)

    gathers = []
    for q in range(NQ):
        dloads[q].wait()
        gathers.append(
            pltpu.async_copy(
                table_hbm.at[idxv.at[pl.ds(q * EQ, EQ)]],
                rowsv.at[pl.ds(q * EQ, EQ)],
                gsems[q],
            )
        )

    zeros16 = lax.iota(jnp.int32, L) * 0
    bias_vec = plsc.load_gather(biasv, [zeros16, zeros16])

    for q in range(NQ):
        gathers[q].wait()

        def reduce_q(c, carry, q=q):
            if q == 0:
                acc = bias_vec + rowsv[pl.ds(c * L, L)]
                flo = 1
            else:
                acc = outv[pl.ds(c * L, L)]
                flo = q * FQ
            for f in range(flo, (q + 1) * FQ):
                acc = acc + rowsv[pl.ds(f * PER_W + c * L, L)]
            outv[pl.ds(c * L, L)] = acc
            return carry

        lax.fori_loop(0, NCHUNK, reduce_q, 0)

    pltpu.sync_copy(outv, out_hbm.at[pl.ds(base, PER_W)])


@jax.jit
def kernel(x, table, bias):
    xt = x.T
    tf = lax.dynamic_update_slice(
        jnp.zeros((TABLE_PAD, 1), jnp.float32), table, (0, 0)
    ).reshape(-1)
    idx_all = _sc_build(xt)
    out = _sc_main(idx_all, tf, bias)
    return out.reshape(B, 1)


# speedup vs baseline: 3.2808x; 1.0004x over previous
import functools

import jax
import jax.numpy as jnp
from jax import lax
from jax.experimental import pallas as pl
from jax.experimental.pallas import tpu as pltpu
from jax.experimental.pallas import tpu_sc as plsc

F = 26
B = 16384
FIELD_DIM = 100000
TABLE_N = F * FIELD_DIM
TABLE_PAD = 2600960
L = 16
NC = 2
NS = 16
NW = NC * NS
PER_W = B // NW
E = PER_W * F
NCHUNK = PER_W // L
FQ = 2

_MESH = plsc.VectorSubcoreMesh(
    core_axis_name="c", subcore_axis_name="s", num_cores=NC, num_subcores=NS
)


def _worker_base():
    return (lax.axis_index("s") * NC + lax.axis_index("c")) * PER_W


@functools.partial(
    pl.kernel,
    mesh=_MESH,
    out_type=jax.ShapeDtypeStruct((B * F,), jnp.int32),
    compiler_params=pltpu.CompilerParams(needs_layout_passes=False),
    scratch_types=[
        pltpu.VMEM((E,), jnp.int32),
        pltpu.SemaphoreType.DMA,
    ],
)
def _sc_build(xt_hbm, idx_hbm, idxv, sem):
    base = _worker_base()

    descs = [
        pltpu.async_copy(
            xt_hbm.at[f, pl.ds(base, PER_W)],
            idxv.at[pl.ds(f * PER_W, PER_W)],
            sem,
        )
        for f in range(F)
    ]
    for d in descs:
        d.wait()

    def build(c, carry):
        off = (c // NCHUNK) * FIELD_DIM
        idxv[pl.ds(c * L, L)] = idxv[pl.ds(c * L, L)] + off
        return carry

    lax.fori_loop(0, E // L, build, 0)
    pltpu.sync_copy(idxv, idx_hbm.at[pl.ds(base * F, E)])


@functools.partial(
    pl.kernel,
    mesh=_MESH,
    out_type=jax.ShapeDtypeStruct((B,), jnp.float32),
    compiler_params=pltpu.CompilerParams(needs_layout_passes=False),
    scratch_types=[
        pltpu.VMEM((E,), jnp.int32),
        pltpu.VMEM((E,), jnp.float32),
        pltpu.VMEM((PER_W,), jnp.float32),
        pltpu.VMEM((1, 1), jnp.float32),
        pltpu.SemaphoreType.DMA,
        [pltpu.SemaphoreType.DMA] * (F // FQ),
    ],
)
def _sc_main(idx_hbm, table_hbm, bias_hbm, out_hbm,
             idxv, rowsv, outv, biasv, sem, gsems):
    base = _worker_base()
    NQ = F // FQ
    EQ = FQ * PER_W

    dloads = [
        pltpu.async_copy(
            idx_hbm.at[pl.ds(base * F + q * EQ, EQ)],
            idxv.at[pl.ds(q * EQ, EQ)],
            sem,
        )
        for q in range(NQ)
    ]
    pltpu.sync_copy(bias_hbm, biasv)

    gathers = []
    for q in range(NQ):
        dloads[q].wait()
        gathers.append(
            pltpu.async_copy(
                table_hbm.at[idxv.at[pl.ds(q * EQ, EQ)]],
                rowsv.at[pl.ds(q * EQ, EQ)],
                gsems[q],
            )
        )

    zeros16 = lax.iota(jnp.int32, L) * 0
    bias_vec = plsc.load_gather(biasv, [zeros16, zeros16])

    for q in range(NQ):
        gathers[q].wait()

        def reduce_q(c, carry, q=q):
            if q == 0:
                acc = bias_vec + rowsv[pl.ds(c * L, L)]
                flo = 1
            else:
                acc = outv[pl.ds(c * L, L)]
                flo = q * FQ
            for f in range(flo, (q + 1) * FQ):
                acc = acc + rowsv[pl.ds(f * PER_W + c * L, L)]
            outv[pl.ds(c * L, L)] = acc
            return carry

        lax.fori_loop(0, NCHUNK, reduce_q, 0)

    pltpu.sync_copy(outv, out_hbm.at[pl.ds(base, PER_W)])


@jax.jit
def kernel(x, table, bias):
    xt = x.T
    tf = lax.dynamic_update_slice(
        jnp.zeros((TABLE_PAD, 1), jnp.float32), table, (0, 0)
    ).reshape(-1)
    idx_all = _sc_build(xt)
    out = _sc_main(idx_all, tf, bias)
    return out.reshape(B, 1)
